# baseline (speedup 1.0000x reference)
import jax
import jax.numpy as jnp
from jax import lax
from jax.experimental import pallas as pl
from jax.experimental.pallas import tpu as pltpu
from jax.experimental.pallas import tpu_sc as plsc

EMBED_DIM = 64
NUM_CORES = 2
NUM_SUBCORES = 16
NW = NUM_CORES * NUM_SUBCORES
CHUNK = 128
K = 2
GROUP = CHUNK * K
GROUP2 = GROUP * EMBED_DIM // 128
NBUF = 4


def _gather_body(idx_hbm, table_hbm, out_hbm, idx_v, rows_v, gsem, osem):
    wid = lax.axis_index("s") * NUM_CORES + lax.axis_index("c")
    nchunk = idx_hbm.shape[0] // NW
    ngroup = nchunk // K
    base = wid * nchunk

    pltpu.sync_copy(idx_hbm.at[pl.ds(base, nchunk)], idx_v)

    def fire(g, b):
        for j in range(K):
            pltpu.async_copy(
                table_hbm.at[idx_v.at[g * K + j]],
                rows_v.at[b].at[pl.ds(j * CHUNK, CHUNK)],
                gsem.at[b])

    def drain(g, b):
        for j in range(K):
            pltpu.make_async_copy(
                table_hbm.at[idx_v.at[g * K + j]],
                rows_v.at[b].at[pl.ds(j * CHUNK, CHUNK)],
                gsem.at[b]).wait()

    def out_copy(g, b):
        return pltpu.make_async_copy(
            rows_v.at[b],
            out_hbm.at[pl.ds(base * CHUNK + g * GROUP, GROUP)],
            osem.at[b])

    for b in range(NBUF):
        fire(b, b)

    def step(p, carry):
        for b in range(NBUF):
            g = p * NBUF + b
            drain(g, b)
            out_copy(g, b).start()
            nxt = g + NBUF

            @pl.when(nxt < ngroup)
            def _():
                out_copy(g, b).wait()
                fire(nxt, b)
        return carry

    lax.fori_loop(0, ngroup // NBUF, step, 0)

    for b in range(NBUF):
        out_copy(ngroup - NBUF + b, b).wait()


def kernel(input_x, table):
    batch, hist = input_x.shape
    n = batch * hist
    idx2d = input_x.astype(jnp.int32).reshape(n // 128, 128)
    mesh = plsc.VectorSubcoreMesh(core_axis_name="c", subcore_axis_name="s")
    out2d = pl.kernel(
        _gather_body,
        out_type=jax.ShapeDtypeStruct((n, EMBED_DIM), jnp.float32),
        mesh=mesh,
        compiler_params=pltpu.CompilerParams(use_tc_tiling_on_sc=False),
        scratch_types=[
            pltpu.VMEM((n // 128 // NW, 128), jnp.int32),
            pltpu.VMEM((NBUF, GROUP, EMBED_DIM), jnp.float32),
            pltpu.SemaphoreType.DMA((NBUF,)),
            pltpu.SemaphoreType.DMA((NBUF,)),
        ],
    )(idx2d, table)
    return out2d.reshape(batch, hist, EMBED_DIM)


# speedup vs baseline: 1.4320x; 1.4320x over previous
import jax
import jax.numpy as jnp
from jax import lax
from jax.experimental import pallas as pl
from jax.experimental.pallas import tpu as pltpu
from jax.experimental.pallas import tpu_sc as plsc

EMBED_DIM = 64
NUM_CORES = 2
NUM_SUBCORES = 16
NW = NUM_CORES * NUM_SUBCORES
CHUNK = 128
K = 2
GROUP = CHUNK * K
NBUF = 4


def _gather_body(idx_hbm, table_hbm, out_hbm, idx_v, rows_v, gsem, osem):
    wid = lax.axis_index("s") * NUM_CORES + lax.axis_index("c")
    nchunk = idx_hbm.shape[0] // NW
    ngroup = nchunk // K
    base = wid * nchunk

    pltpu.sync_copy(idx_hbm.at[pl.ds(base, nchunk)], idx_v)

    def fire(g, b):
        for j in range(K):
            pltpu.async_copy(
                table_hbm.at[idx_v.at[g * K + j]],
                rows_v.at[b].at[pl.ds(j * CHUNK, CHUNK)],
                gsem.at[b])

    def drain(g, b):
        for j in range(K):
            pltpu.make_async_copy(
                table_hbm.at[idx_v.at[g * K + j]],
                rows_v.at[b].at[pl.ds(j * CHUNK, CHUNK)],
                gsem.at[b]).wait()

    def out_copy(g, b):
        return pltpu.make_async_copy(
            rows_v.at[b],
            out_hbm.at[pl.ds(base * CHUNK + g * GROUP, GROUP),
                       pl.ds(0, EMBED_DIM)],
            osem.at[b])

    for b in range(NBUF):
        fire(b, b)

    def step(p, carry):
        for b in range(NBUF):
            g = p * NBUF + b
            drain(g, b)
            out_copy(g, b).start()
            nxt = g + NBUF

            @pl.when(nxt < ngroup)
            def _():
                out_copy(g, b).wait()
                fire(nxt, b)
        return carry

    lax.fori_loop(0, ngroup // NBUF, step, 0)

    for b in range(NBUF):
        out_copy(ngroup - NBUF + b, b).wait()


def kernel(input_x, table):
    batch, hist = input_x.shape
    n = batch * hist
    idx2d = (input_x.astype(jnp.int32) * 2).reshape(n // 128, 128)
    tpad = jnp.pad(table, ((0, 0), (0, 128 - EMBED_DIM)))
    t2 = tpad.reshape(2 * table.shape[0], EMBED_DIM)
    mesh = plsc.VectorSubcoreMesh(core_axis_name="c", subcore_axis_name="s")
    out128 = pl.kernel(
        _gather_body,
        out_type=jax.ShapeDtypeStruct((n, 128), jnp.float32),
        mesh=mesh,
        compiler_params=pltpu.CompilerParams(use_tc_tiling_on_sc=False),
        scratch_types=[
            pltpu.VMEM((n // 128 // NW, 128), jnp.int32),
            pltpu.VMEM((NBUF, GROUP, EMBED_DIM), jnp.float32),
            pltpu.SemaphoreType.DMA((NBUF,)),
            pltpu.SemaphoreType.DMA((NBUF,)),
        ],
    )(idx2d, t2)
    out = jax.lax.slice(out128, (0, 0), (n, EMBED_DIM))
    return out.reshape(batch, hist, EMBED_DIM)
